# Initial kernel scaffold; baseline (speedup 1.0000x reference)
import functools

import jax
import jax.numpy as jnp
from jax import lax
from jax.experimental import pallas as pl
from jax.experimental.pallas import tpu as pltpu
from jax.experimental.pallas import tpu_sc as plsc

N_NODES = 10000
N_EDGES = 320000
D = 128

NC = 2
NS = 16
NW = NC * NS
CHUNK = 128
E_PAD = 327680
CHUNKS_PER_W = E_PAD // (NW * CHUNK)
N_PAD = 10240
ROWS_PER_TILE = N_PAD // NS


def _sc_aggregate(x_pad, src3, dst3, w3, ones3):
  mesh = plsc.VectorSubcoreMesh(core_axis_name="c", subcore_axis_name="s")

  @functools.partial(
      pl.kernel,
      out_type=[
          jax.ShapeDtypeStruct((NC, N_PAD, D), jnp.float32),
          jax.ShapeDtypeStruct((NC, N_PAD), jnp.float32),
      ],
      mesh=mesh,
      scratch_types=[
          pltpu.VMEM((CHUNK,), jnp.int32),
          pltpu.VMEM((CHUNK,), jnp.int32),
          pltpu.VMEM((CHUNK,), jnp.float32),
          pltpu.VMEM((CHUNK,), jnp.float32),
          pltpu.VMEM((CHUNK, D), jnp.float32),
          pltpu.VMEM((CHUNK, D), jnp.float32),
          pltpu.VMEM((ROWS_PER_TILE,), jnp.float32),
          pltpu.VMEM_SHARED((N_PAD, D), jnp.float32),
          pltpu.VMEM_SHARED((N_PAD,), jnp.float32),
          pltpu.SemaphoreType.DMA,
      ],
  )
  def agg(x_hbm, src_hbm, dst_hbm, w_hbm, ones_hbm, part_out, deg_out,
          sidx, didx, wv, onesv, rows, zblk, zdeg, acc, dacc, sem):
    cid = lax.axis_index("c")
    sid = lax.axis_index("s")
    wid = cid * NS + sid
    zero16 = jnp.zeros((16,), jnp.float32)

    def fill_zero(r, carry):
      for c in range(D // 16):
        zblk[r, pl.ds(c * 16, 16)] = zero16
      return carry
    lax.fori_loop(0, CHUNK, fill_zero, 0)
    for j in range(ROWS_PER_TILE // 16):
      zdeg[pl.ds(j * 16, 16)] = zero16
    row0 = sid * ROWS_PER_TILE
    for j in range(ROWS_PER_TILE // CHUNK):
      pltpu.sync_copy(zblk, acc.at[pl.ds(row0 + j * CHUNK, CHUNK)])
    pltpu.sync_copy(zdeg, dacc.at[pl.ds(row0, ROWS_PER_TILE)])
    plsc.subcore_barrier()

    def chunk_body(k, carry):
      pltpu.sync_copy(src_hbm.at[wid, k], sidx)
      pltpu.sync_copy(dst_hbm.at[wid, k], didx)
      pltpu.sync_copy(w_hbm.at[wid, k], wv)
      pltpu.sync_copy(ones_hbm.at[wid, k], onesv)
      pltpu.async_copy(x_hbm.at[sidx], rows, sem).wait()

      def scale_body(i, c2):
        w = wv[i]
        for c in range(D // 16):
          rows[i, pl.ds(c * 16, 16)] = rows[i, pl.ds(c * 16, 16)] * w
        return c2
      lax.fori_loop(0, CHUNK, scale_body, 0)

      pltpu.sync_copy(rows, acc.at[didx], add=True)
      pltpu.sync_copy(onesv, dacc.at[didx], add=True)
      return carry
    lax.fori_loop(0, CHUNKS_PER_W, chunk_body, 0)
    plsc.subcore_barrier()

    pltpu.sync_copy(acc.at[pl.ds(row0, ROWS_PER_TILE)],
                    part_out.at[cid, pl.ds(row0, ROWS_PER_TILE)])
    pltpu.sync_copy(dacc.at[pl.ds(row0, ROWS_PER_TILE)],
                    deg_out.at[cid, pl.ds(row0, ROWS_PER_TILE)])

  return agg(x_pad, src3, dst3, w3, ones3)


def _tc_combine(x_pad, p0, p1, d0, d1, W_self, W_neigh, bias):
  BLK = 1024
  grid = (N_PAD // BLK,)

  def body(x_ref, p0_ref, p1_ref, d0_ref, d1_ref, ws_ref, wn_ref, b_ref,
           o_ref):
    deg = d0_ref[...] + d1_ref[...]
    neigh = (p0_ref[...] + p1_ref[...]) / jnp.maximum(deg, 1.0)
    rst = (
        jnp.dot(x_ref[...], ws_ref[...], preferred_element_type=jnp.float32)
        + jnp.dot(neigh, wn_ref[...], preferred_element_type=jnp.float32)
        + b_ref[...]
    )
    o_ref[...] = jnp.where(rst >= 0, rst, 0.01 * rst)

  return pl.pallas_call(
      body,
      grid=grid,
      in_specs=[
          pl.BlockSpec((BLK, D), lambda i: (i, 0)),
          pl.BlockSpec((BLK, D), lambda i: (i, 0)),
          pl.BlockSpec((BLK, D), lambda i: (i, 0)),
          pl.BlockSpec((BLK, 1), lambda i: (i, 0)),
          pl.BlockSpec((BLK, 1), lambda i: (i, 0)),
          pl.BlockSpec((D, D), lambda i: (0, 0)),
          pl.BlockSpec((D, D), lambda i: (0, 0)),
          pl.BlockSpec((1, D), lambda i: (0, 0)),
      ],
      out_specs=pl.BlockSpec((BLK, D), lambda i: (i, 0)),
      out_shape=jax.ShapeDtypeStruct((N_PAD, D), jnp.float32),
  )(x_pad, p0, p1, d0, d1, W_self, W_neigh, bias)


def kernel(node_embeddings, edge_index, edge_weight, W_self, W_neigh, bias):
  src = edge_index[0].astype(jnp.int32)
  dst = edge_index[1].astype(jnp.int32)
  w = edge_weight.astype(jnp.float32)

  pad = E_PAD - N_EDGES
  pad_idx = (jnp.arange(pad, dtype=jnp.int32) % N_NODES)
  src_p = jnp.concatenate([src, pad_idx])
  dst_p = jnp.concatenate([dst, pad_idx])
  w_p = jnp.concatenate([w, jnp.zeros((pad,), jnp.float32)])
  ones_p = jnp.concatenate(
      [jnp.ones((N_EDGES,), jnp.float32), jnp.zeros((pad,), jnp.float32)])

  shape3 = (NW, CHUNKS_PER_W, CHUNK)
  src3 = src_p.reshape(shape3)
  dst3 = dst_p.reshape(shape3)
  w3 = w_p.reshape(shape3)
  ones3 = ones_p.reshape(shape3)

  x_pad = jnp.pad(node_embeddings, ((0, N_PAD - N_NODES), (0, 0)))

  partials, degs = _sc_aggregate(x_pad, src3, dst3, w3, ones3)

  out_pad = _tc_combine(
      x_pad,
      partials[0], partials[1],
      degs[0].reshape(N_PAD, 1), degs[1].reshape(N_PAD, 1),
      W_self, W_neigh,
      bias.reshape(1, D),
  )
  return out_pad[:N_NODES]


# speedup vs baseline: 5.0186x; 5.0186x over previous
import functools

import jax
import jax.numpy as jnp
from jax import lax
from jax.experimental import pallas as pl
from jax.experimental.pallas import tpu as pltpu
from jax.experimental.pallas import tpu_sc as plsc

N_NODES = 10000
N_EDGES = 320000
D = 128

NC = 2
NS = 16
NW = NC * NS
CHUNK = 128
E_PAD = 327680
CHUNKS_PER_W = E_PAD // (NW * CHUNK)
N_PAD = 10240
ROWS_PER_TILE = N_PAD // NS


def _sc_aggregate(x_pad, src3, dst3, w3, ones3):
  mesh = plsc.VectorSubcoreMesh(core_axis_name="c", subcore_axis_name="s")

  @functools.partial(
      pl.kernel,
      out_type=[
          jax.ShapeDtypeStruct((NC, N_PAD, D), jnp.float32),
          jax.ShapeDtypeStruct((NC, N_PAD), jnp.float32),
      ],
      mesh=mesh,
      scratch_types=[
          pltpu.VMEM((CHUNK,), jnp.int32),
          pltpu.VMEM((CHUNK,), jnp.int32),
          pltpu.VMEM((CHUNK,), jnp.float32),
          pltpu.VMEM((CHUNK,), jnp.float32),
          pltpu.VMEM((CHUNK, D), jnp.float32),
          pltpu.VMEM((CHUNK, D), jnp.float32),
          pltpu.VMEM((ROWS_PER_TILE,), jnp.float32),
          pltpu.VMEM_SHARED((N_PAD, D), jnp.float32),
          pltpu.VMEM_SHARED((N_PAD,), jnp.float32),
          pltpu.SemaphoreType.DMA,
      ],
  )
  def agg(x_hbm, src_hbm, dst_hbm, w_hbm, ones_hbm, part_out, deg_out,
          sidx, didx, wv, onesv, rows, zblk, zdeg, acc, dacc, sem):
    cid = lax.axis_index("c")
    sid = lax.axis_index("s")
    wid = cid * NS + sid
    zero16 = jnp.zeros((16,), jnp.float32)

    def fill_zero(r, carry):
      for c in range(D // 16):
        zblk[r, pl.ds(c * 16, 16)] = zero16
      return carry
    lax.fori_loop(0, CHUNK, fill_zero, 0)
    for j in range(ROWS_PER_TILE // 16):
      zdeg[pl.ds(j * 16, 16)] = zero16
    row0 = sid * ROWS_PER_TILE
    for j in range(ROWS_PER_TILE // CHUNK):
      pltpu.sync_copy(zblk, acc.at[pl.ds(row0 + j * CHUNK, CHUNK)])
    pltpu.sync_copy(zdeg, dacc.at[pl.ds(row0, ROWS_PER_TILE)])
    plsc.subcore_barrier()

    def chunk_body(k, carry):
      pltpu.sync_copy(src_hbm.at[wid, k], sidx)
      pltpu.sync_copy(dst_hbm.at[wid, k], didx)
      pltpu.sync_copy(w_hbm.at[wid, k], wv)
      pltpu.sync_copy(ones_hbm.at[wid, k], onesv)
      pltpu.async_copy(x_hbm.at[sidx], rows, sem).wait()

      def scale_group(g, c2):
        wg = wv[pl.ds(g * 16, 16)]
        for j in range(16):
          w = wg[j]
          i = g * 16 + j
          for c in range(D // 16):
            rows[i, pl.ds(c * 16, 16)] = rows[i, pl.ds(c * 16, 16)] * w
        return c2
      lax.fori_loop(0, CHUNK // 16, scale_group, 0)

      pltpu.sync_copy(rows, acc.at[didx], add=True)
      pltpu.sync_copy(onesv, dacc.at[didx], add=True)
      return carry
    lax.fori_loop(0, CHUNKS_PER_W, chunk_body, 0)
    plsc.subcore_barrier()

    pltpu.sync_copy(acc.at[pl.ds(row0, ROWS_PER_TILE)],
                    part_out.at[cid, pl.ds(row0, ROWS_PER_TILE)])
    pltpu.sync_copy(dacc.at[pl.ds(row0, ROWS_PER_TILE)],
                    deg_out.at[cid, pl.ds(row0, ROWS_PER_TILE)])

  return agg(x_pad, src3, dst3, w3, ones3)


def _tc_combine(x_pad, p0, p1, d0, d1, W_self, W_neigh, bias):
  BLK = 1024
  grid = (N_PAD // BLK,)

  def body(x_ref, p0_ref, p1_ref, d0_ref, d1_ref, ws_ref, wn_ref, b_ref,
           o_ref):
    deg = d0_ref[...] + d1_ref[...]
    neigh = (p0_ref[...] + p1_ref[...]) / jnp.maximum(deg, 1.0)
    rst = (
        jnp.dot(x_ref[...], ws_ref[...], preferred_element_type=jnp.float32)
        + jnp.dot(neigh, wn_ref[...], preferred_element_type=jnp.float32)
        + b_ref[...]
    )
    o_ref[...] = jnp.where(rst >= 0, rst, 0.01 * rst)

  return pl.pallas_call(
      body,
      grid=grid,
      in_specs=[
          pl.BlockSpec((BLK, D), lambda i: (i, 0)),
          pl.BlockSpec((BLK, D), lambda i: (i, 0)),
          pl.BlockSpec((BLK, D), lambda i: (i, 0)),
          pl.BlockSpec((BLK, 1), lambda i: (i, 0)),
          pl.BlockSpec((BLK, 1), lambda i: (i, 0)),
          pl.BlockSpec((D, D), lambda i: (0, 0)),
          pl.BlockSpec((D, D), lambda i: (0, 0)),
          pl.BlockSpec((1, D), lambda i: (0, 0)),
      ],
      out_specs=pl.BlockSpec((BLK, D), lambda i: (i, 0)),
      out_shape=jax.ShapeDtypeStruct((N_PAD, D), jnp.float32),
  )(x_pad, p0, p1, d0, d1, W_self, W_neigh, bias)


def kernel(node_embeddings, edge_index, edge_weight, W_self, W_neigh, bias):
  src = edge_index[0].astype(jnp.int32)
  dst = edge_index[1].astype(jnp.int32)
  w = edge_weight.astype(jnp.float32)

  pad = E_PAD - N_EDGES
  pad_idx = (jnp.arange(pad, dtype=jnp.int32) % N_NODES)
  src_p = jnp.concatenate([src, pad_idx])
  dst_p = jnp.concatenate([dst, pad_idx])
  w_p = jnp.concatenate([w, jnp.zeros((pad,), jnp.float32)])
  ones_p = jnp.concatenate(
      [jnp.ones((N_EDGES,), jnp.float32), jnp.zeros((pad,), jnp.float32)])

  shape3 = (NW, CHUNKS_PER_W, CHUNK)
  src3 = src_p.reshape(shape3)
  dst3 = dst_p.reshape(shape3)
  w3 = w_p.reshape(shape3)
  ones3 = ones_p.reshape(shape3)

  x_pad = jnp.pad(node_embeddings, ((0, N_PAD - N_NODES), (0, 0)))

  partials, degs = _sc_aggregate(x_pad, src3, dst3, w3, ones3)

  out_pad = _tc_combine(
      x_pad,
      partials[0], partials[1],
      degs[0].reshape(N_PAD, 1), degs[1].reshape(N_PAD, 1),
      W_self, W_neigh,
      bias.reshape(1, D),
  )
  return out_pad[:N_NODES]


# speedup vs baseline: 9.4172x; 1.8764x over previous
import functools

import jax
import jax.numpy as jnp
from jax import lax
from jax.experimental import pallas as pl
from jax.experimental.pallas import tpu as pltpu
from jax.experimental.pallas import tpu_sc as plsc

N_NODES = 10000
N_EDGES = 320000
D = 128

NC = 2
NS = 16
NW = NC * NS
CHUNK = 128
E_PAD = 327680
CHUNKS_PER_W = E_PAD // (NW * CHUNK)
N_PAD = 10240
ROWS_PER_TILE = N_PAD // NS


def _sc_aggregate(x_pad, idx4, wf4):
  mesh = plsc.VectorSubcoreMesh(core_axis_name="c", subcore_axis_name="s")
  K = CHUNKS_PER_W

  @functools.partial(
      pl.kernel,
      out_type=[
          jax.ShapeDtypeStruct((NC, N_PAD, D), jnp.float32),
          jax.ShapeDtypeStruct((NC, N_PAD), jnp.float32),
      ],
      mesh=mesh,
      scratch_types=[
          pltpu.VMEM((2, 2, CHUNK), jnp.int32),
          pltpu.VMEM((2, 2, CHUNK), jnp.float32),
          pltpu.VMEM((CHUNK, D), jnp.float32),
          pltpu.VMEM((CHUNK, D), jnp.float32),
          pltpu.VMEM((ROWS_PER_TILE,), jnp.float32),
          pltpu.VMEM_SHARED((N_PAD, D), jnp.float32),
          pltpu.VMEM_SHARED((N_PAD,), jnp.float32),
          pltpu.SemaphoreType.DMA,
          pltpu.SemaphoreType.DMA,
          pltpu.SemaphoreType.DMA,
          pltpu.SemaphoreType.DMA,
          pltpu.SemaphoreType.DMA,
          pltpu.SemaphoreType.DMA,
          pltpu.SemaphoreType.DMA,
          pltpu.SemaphoreType.DMA,
      ],
  )
  def agg(x_hbm, idx_hbm, wf_hbm, part_out, deg_out,
          idxb, wfb, rows0, rows1, zdeg, acc, dacc,
          sm0, sm1, sg0, sg1, ss0, ss1, sd0, sd1):
    cid = lax.axis_index("c")
    sid = lax.axis_index("s")
    wid = cid * NS + sid
    zero16 = jnp.zeros((16,), jnp.float32)
    rows = (rows0, rows1)
    sm = (sm0, sm1)
    sg = (sg0, sg1)
    ss = (ss0, ss1)
    sd = (sd0, sd1)

    pltpu.sync_copy(idx_hbm.at[wid, 0], idxb.at[0])
    pltpu.sync_copy(wf_hbm.at[wid, 0], wfb.at[0])
    pltpu.async_copy(x_hbm.at[idxb.at[0, 0]], rows0, sg0)

    def fill_zero(r, carry):
      for c in range(D // 16):
        rows1[r, pl.ds(c * 16, 16)] = zero16
      return carry
    lax.fori_loop(0, CHUNK, fill_zero, 0)
    for j in range(ROWS_PER_TILE // 16):
      zdeg[pl.ds(j * 16, 16)] = zero16
    row0 = sid * ROWS_PER_TILE
    for j in range(ROWS_PER_TILE // CHUNK):
      pltpu.sync_copy(rows1, acc.at[pl.ds(row0 + j * CHUNK, CHUNK)])
    pltpu.sync_copy(zdeg, dacc.at[pl.ds(row0, ROWS_PER_TILE)])
    plsc.subcore_barrier()

    def meta_descs(k, b):
      i = pltpu.make_async_copy(idx_hbm.at[wid, k], idxb.at[b], sm[b])
      f = pltpu.make_async_copy(wf_hbm.at[wid, k], wfb.at[b], sm[b])
      return i, f

    def scatter_descs(k, b):
      s = pltpu.make_async_copy(rows[b], acc.at[idxb.at[b, 1]], ss[b])
      d = pltpu.make_async_copy(wfb.at[b, 1], dacc.at[idxb.at[b, 1]], sd[b])
      return s, d

    def scale(b):
      def scale_group(g, c2):
        wg = wfb[b, 0, pl.ds(g * 16, 16)]
        for j in range(16):
          w = wg[j]
          i = g * 16 + j
          for c in range(D // 16):
            rows[b][i, pl.ds(c * 16, 16)] = rows[b][i, pl.ds(c * 16, 16)] * w
        return c2
      lax.fori_loop(0, CHUNK // 16, scale_group, 0)

    @pl.loop(0, K, step=2)
    def pipeline(k0):
      for b in (0, 1):
        k = k0 + b
        nb = 1 - b
        if b == 0:
          @pl.when(k0 > 0)
          def _():
            s, d = scatter_descs(k0 - 1, 1)
            s.wait()
            d.wait()
        else:
          s, d = scatter_descs(k0, 0)
          s.wait()
          d.wait()
        if b == 0:
          i_d, f_d = meta_descs(k + 1, 1)
          i_d.start()
          f_d.start()
        else:
          @pl.when(k0 + 2 < K)
          def _():
            i_d, f_d = meta_descs(k + 1, 0)
            i_d.start()
            f_d.start()
        pltpu.make_async_copy(x_hbm.at[idxb.at[b, 0]], rows[b], sg[b]).wait()
        if b == 0:
          i_d, f_d = meta_descs(k + 1, 1)
          i_d.wait()
          f_d.wait()
          pltpu.async_copy(x_hbm.at[idxb.at[1, 0]], rows[1], sg[1])
        else:
          @pl.when(k0 + 2 < K)
          def _():
            i_d, f_d = meta_descs(k + 1, 0)
            i_d.wait()
            f_d.wait()
            pltpu.async_copy(x_hbm.at[idxb.at[0, 0]], rows[0], sg[0])
        scale(b)
        pltpu.async_copy(rows[b], acc.at[idxb.at[b, 1]], ss[b], add=True)
        pltpu.async_copy(wfb.at[b, 1], dacc.at[idxb.at[b, 1]], sd[b],
                         add=True)

    s, d = scatter_descs(K - 1, 1)
    s.wait()
    d.wait()
    plsc.subcore_barrier()

    pltpu.sync_copy(acc.at[pl.ds(row0, ROWS_PER_TILE)],
                    part_out.at[cid, pl.ds(row0, ROWS_PER_TILE)])
    pltpu.sync_copy(dacc.at[pl.ds(row0, ROWS_PER_TILE)],
                    deg_out.at[cid, pl.ds(row0, ROWS_PER_TILE)])

  return agg(x_pad, idx4, wf4)


def _tc_combine(x_pad, p0, p1, d0, d1, W_self, W_neigh, bias):
  BLK = 1024
  grid = (N_PAD // BLK,)

  def body(x_ref, p0_ref, p1_ref, d0_ref, d1_ref, ws_ref, wn_ref, b_ref,
           o_ref):
    deg = d0_ref[...] + d1_ref[...]
    neigh = (p0_ref[...] + p1_ref[...]) / jnp.maximum(deg, 1.0)
    rst = (
        jnp.dot(x_ref[...], ws_ref[...], preferred_element_type=jnp.float32)
        + jnp.dot(neigh, wn_ref[...], preferred_element_type=jnp.float32)
        + b_ref[...]
    )
    o_ref[...] = jnp.where(rst >= 0, rst, 0.01 * rst)

  return pl.pallas_call(
      body,
      grid=grid,
      in_specs=[
          pl.BlockSpec((BLK, D), lambda i: (i, 0)),
          pl.BlockSpec((BLK, D), lambda i: (i, 0)),
          pl.BlockSpec((BLK, D), lambda i: (i, 0)),
          pl.BlockSpec((BLK, 1), lambda i: (i, 0)),
          pl.BlockSpec((BLK, 1), lambda i: (i, 0)),
          pl.BlockSpec((D, D), lambda i: (0, 0)),
          pl.BlockSpec((D, D), lambda i: (0, 0)),
          pl.BlockSpec((1, D), lambda i: (0, 0)),
      ],
      out_specs=pl.BlockSpec((BLK, D), lambda i: (i, 0)),
      out_shape=jax.ShapeDtypeStruct((N_PAD, D), jnp.float32),
  )(x_pad, p0, p1, d0, d1, W_self, W_neigh, bias)


def kernel(node_embeddings, edge_index, edge_weight, W_self, W_neigh, bias):
  src = edge_index[0].astype(jnp.int32)
  dst = edge_index[1].astype(jnp.int32)
  w = edge_weight.astype(jnp.float32)

  pad = E_PAD - N_EDGES
  pad_idx = (jnp.arange(pad, dtype=jnp.int32) % N_NODES)
  src_p = jnp.concatenate([src, pad_idx])
  dst_p = jnp.concatenate([dst, pad_idx])
  w_p = jnp.concatenate([w, jnp.zeros((pad,), jnp.float32)])
  ones_p = jnp.concatenate(
      [jnp.ones((N_EDGES,), jnp.float32), jnp.zeros((pad,), jnp.float32)])

  shape3 = (NW, CHUNKS_PER_W, CHUNK)
  idx4 = jnp.stack([src_p.reshape(shape3), dst_p.reshape(shape3)], axis=2)
  wf4 = jnp.stack([w_p.reshape(shape3), ones_p.reshape(shape3)], axis=2)

  x_pad = jnp.pad(node_embeddings, ((0, N_PAD - N_NODES), (0, 0)))

  partials, degs = _sc_aggregate(x_pad, idx4, wf4)

  out_pad = _tc_combine(
      x_pad,
      partials[0], partials[1],
      degs[0].reshape(N_PAD, 1), degs[1].reshape(N_PAD, 1),
      W_self, W_neigh,
      bias.reshape(1, D),
  )
  return out_pad[:N_NODES]


# speedup vs baseline: 11.9174x; 1.2655x over previous
import functools

import jax
import jax.numpy as jnp
from jax import lax
from jax.experimental import pallas as pl
from jax.experimental.pallas import tpu as pltpu
from jax.experimental.pallas import tpu_sc as plsc

N_NODES = 10000
N_EDGES = 320000
D = 128

NC = 2
NS = 16
NW = NC * NS
CHUNK = 112
HA, HB = 64, 48
E_PER_W = N_EDGES // NW
K_FULL = E_PER_W // CHUNK
TAIL = E_PER_W - K_FULL * CHUNK
N_PAD = 10240
ROWS_PER_TILE = N_PAD // NS


def _sc_aggregate(x, src_idx, dst_idx, edge_weight):
  mesh = plsc.VectorSubcoreMesh(core_axis_name="c", subcore_axis_name="s")
  K = K_FULL

  @functools.partial(
      pl.kernel,
      out_type=[
          jax.ShapeDtypeStruct((N_PAD, D), jnp.float32),
          jax.ShapeDtypeStruct((N_PAD, D), jnp.float32),
          jax.ShapeDtypeStruct((N_PAD,), jnp.float32),
          jax.ShapeDtypeStruct((N_PAD,), jnp.float32),
      ],
      mesh=mesh,
      scratch_types=[
          pltpu.VMEM((6, CHUNK), jnp.int32),
          pltpu.VMEM((6, HA), jnp.int32),
          pltpu.VMEM((6, HB), jnp.int32),
          pltpu.VMEM((6, CHUNK), jnp.float32),
          pltpu.VMEM((HA,), jnp.float32),
          pltpu.VMEM((CHUNK, D), jnp.float32),
          pltpu.VMEM((CHUNK, D), jnp.float32),
          pltpu.VMEM((CHUNK, D), jnp.float32),
          pltpu.VMEM((TAIL,), jnp.int32),
          pltpu.VMEM((TAIL,), jnp.int32),
          pltpu.VMEM((ROWS_PER_TILE,), jnp.float32),
          pltpu.VMEM_SHARED((N_PAD, D), jnp.float32),
          pltpu.VMEM_SHARED((N_PAD,), jnp.float32),
          [pltpu.SemaphoreType.DMA] * 6,
          [pltpu.SemaphoreType.DMA] * 3,
          [pltpu.SemaphoreType.DMA] * 3,
          [pltpu.SemaphoreType.DMA] * 3,
      ],
  )
  def agg(x_hbm, si_hbm, di_hbm, ew_hbm, p0_out, p1_out, d0_out, d1_out,
          sidxb, didxa, didxc, wb, ones_v, rows0, rows1, rows2,
          sidx_t, didx_t, zdeg, acc, dacc, sm, sg, ss, sd):
    cid = lax.axis_index("c")
    sid = lax.axis_index("s")
    wid = cid * NS + sid
    e_base = wid * E_PER_W
    zero16 = jnp.zeros((16,), jnp.float32)
    one16 = jnp.ones((16,), jnp.float32)
    rows = (rows0, rows1, rows2)

    def meta_descs(k, bm):
      e0 = e_base + k * CHUNK
      s = pltpu.make_async_copy(si_hbm.at[pl.ds(e0, CHUNK)],
                                sidxb.at[bm], sm[bm])
      d0 = pltpu.make_async_copy(di_hbm.at[pl.ds(N_EDGES + e0, HA)],
                                 didxa.at[bm], sm[bm])
      d1 = pltpu.make_async_copy(di_hbm.at[pl.ds(N_EDGES + e0 + HA, HB)],
                                 didxc.at[bm], sm[bm])
      w = pltpu.make_async_copy(ew_hbm.at[pl.ds(e0, CHUNK)], wb.at[bm],
                                sm[bm])
      return s, d0, d1, w

    def gather_desc(br, bm):
      return pltpu.make_async_copy(x_hbm.at[sidxb.at[bm]], rows[br], sg[br])

    def row_scatter_descs(br, bm):
      s0 = pltpu.make_async_copy(rows[br].at[pl.ds(0, HA)],
                                 acc.at[didxa.at[bm]], ss[br])
      s1 = pltpu.make_async_copy(rows[br].at[pl.ds(HA, HB)],
                                 acc.at[didxc.at[bm]], ss[br])
      return s0, s1

    def deg_scatter_descs(br, bm):
      d0 = pltpu.make_async_copy(ones_v.at[pl.ds(0, HA)],
                                 dacc.at[didxa.at[bm]], sd[br])
      d1 = pltpu.make_async_copy(ones_v.at[pl.ds(0, HB)],
                                 dacc.at[didxc.at[bm]], sd[br])
      return d0, d1

    def scatter_descs(br, bm):
      return row_scatter_descs(br, bm) + deg_scatter_descs(br, bm)

    for desc in meta_descs(0, 0) + meta_descs(1, 1):
      desc.start()
    for desc in meta_descs(0, 0) + meta_descs(1, 1):
      desc.wait()
    gather_desc(0, 0).start()
    gather_desc(1, 1).start()

    for j in range(HA // 16):
      ones_v[pl.ds(j * 16, 16)] = one16

    def fill_zero(r, carry):
      for c in range(D // 16):
        rows2[r, pl.ds(c * 16, 16)] = zero16
      return carry
    lax.fori_loop(0, CHUNK, fill_zero, 0)
    for j in range(ROWS_PER_TILE // 16):
      zdeg[pl.ds(j * 16, 16)] = zero16
    row0 = sid * ROWS_PER_TILE
    nz = ROWS_PER_TILE // CHUNK
    for j in range(nz):
      pltpu.sync_copy(rows2, acc.at[pl.ds(row0 + j * CHUNK, CHUNK)])
    rem = ROWS_PER_TILE - nz * CHUNK
    if rem:
      pltpu.sync_copy(rows2.at[pl.ds(0, rem)],
                      acc.at[pl.ds(row0 + nz * CHUNK, rem)])
    pltpu.sync_copy(zdeg, dacc.at[pl.ds(row0, ROWS_PER_TILE)])
    plsc.subcore_barrier()

    def scale(br, bm, lo, hi):
      def scale_group(g, c2):
        wg = wb[bm, pl.ds(g * 16, 16)]
        for j in range(16):
          w = wg[j]
          i = g * 16 + j
          for c in range(D // 16):
            rows[br][i, pl.ds(c * 16, 16)] = (
                rows[br][i, pl.ds(c * 16, 16)] * w)
        return c2
      lax.fori_loop(lo // 16, hi // 16, scale_group, 0)

    def chunk_step(k, first=False, prefetch=True):
      br, bm = k % 3, k % 6
      brp, bmp = (k + 2) % 3, (k + 2) % 6
      if prefetch:
        for desc in meta_descs(k + 2, bmp):
          desc.start()
      for desc in deg_scatter_descs(br, bm):
        desc.start(add=True)
      gather_desc(br, bm).wait()
      scale(br, bm, 0, HA)
      row_scatter_descs(br, bm)[0].start(add=True)
      scale(br, bm, HA, CHUNK)
      row_scatter_descs(br, bm)[1].start(add=True)
      if not first:
        for desc in scatter_descs((k - 1) % 3, (k - 1) % 6):
          desc.wait()
      if prefetch:
        for desc in meta_descs(k + 2, bmp):
          desc.wait()
        gather_desc(brp, bmp).start()

    chunk_step(0, first=True)
    chunk_step(1)
    chunk_step(2)

    @pl.loop(3, 87, step=6)
    def pipeline(k0):
      for j in range(6):
        k = k0 + j
        br, bm = (3 + j) % 3, (3 + j) % 6
        brp, bmp = (3 + j + 2) % 3, (3 + j + 2) % 6
        for desc in meta_descs(k + 2, bmp):
          desc.start()
        for desc in deg_scatter_descs(br, bm):
          desc.start(add=True)
        gather_desc(br, bm).wait()
        scale(br, bm, 0, HA)
        row_scatter_descs(br, bm)[0].start(add=True)
        scale(br, bm, HA, CHUNK)
        row_scatter_descs(br, bm)[1].start(add=True)
        for desc in scatter_descs((3 + j - 1) % 3, (3 + j - 1) % 6):
          desc.wait()
        for desc in meta_descs(k + 2, bmp):
          desc.wait()
        gather_desc(brp, bmp).start()

    chunk_step(87, prefetch=False)
    chunk_step(88, prefetch=False)

    for desc in scatter_descs(88 % 3, 88 % 6):
      desc.wait()

    e0 = e_base + K * CHUNK
    pltpu.sync_copy(si_hbm.at[pl.ds(e0, TAIL)], sidx_t)
    pltpu.sync_copy(di_hbm.at[pl.ds(N_EDGES + e0, TAIL)], didx_t)
    pltpu.sync_copy(ew_hbm.at[pl.ds(e0, TAIL)], wb.at[0, pl.ds(0, TAIL)])
    pltpu.async_copy(x_hbm.at[sidx_t], rows0.at[pl.ds(0, TAIL)],
                     sg[0]).wait()
    for g in range(TAIL // 16):
      wg = wb[0, pl.ds(g * 16, 16)]
      for j in range(16):
        w = wg[j]
        i = g * 16 + j
        for c in range(D // 16):
          rows0[i, pl.ds(c * 16, 16)] = rows0[i, pl.ds(c * 16, 16)] * w
    pltpu.sync_copy(rows0.at[pl.ds(0, TAIL)], acc.at[didx_t], add=True)
    pltpu.sync_copy(ones_v.at[pl.ds(0, TAIL)], dacc.at[didx_t], add=True)
    plsc.subcore_barrier()

    @pl.when(cid == 0)
    def _():
      pltpu.sync_copy(acc.at[pl.ds(row0, ROWS_PER_TILE)],
                      p0_out.at[pl.ds(row0, ROWS_PER_TILE)])
      pltpu.sync_copy(dacc.at[pl.ds(row0, ROWS_PER_TILE)],
                      d0_out.at[pl.ds(row0, ROWS_PER_TILE)])

    @pl.when(cid == 1)
    def _():
      pltpu.sync_copy(acc.at[pl.ds(row0, ROWS_PER_TILE)],
                      p1_out.at[pl.ds(row0, ROWS_PER_TILE)])
      pltpu.sync_copy(dacc.at[pl.ds(row0, ROWS_PER_TILE)],
                      d1_out.at[pl.ds(row0, ROWS_PER_TILE)])

  return agg(x, src_idx, dst_idx, edge_weight)


BLK = 2000


def _tc_self(x, W_self, bias):
  def body(x_ref, ws_ref, b_ref, o_ref):
    o_ref[...] = (
        jnp.dot(x_ref[...], ws_ref[...], preferred_element_type=jnp.float32)
        + b_ref[...]
    )

  return pl.pallas_call(
      body,
      grid=(N_NODES // BLK,),
      in_specs=[
          pl.BlockSpec((BLK, D), lambda i: (i, 0)),
          pl.BlockSpec((D, D), lambda i: (0, 0)),
          pl.BlockSpec((1, D), lambda i: (0, 0)),
      ],
      out_specs=pl.BlockSpec((BLK, D), lambda i: (i, 0)),
      out_shape=jax.ShapeDtypeStruct((N_NODES, D), jnp.float32),
  )(x, W_self, bias)


def _tc_combine(selfterm, p0, p1, d0, d1, W_neigh):
  def body(s_ref, p0_ref, p1_ref, d0_ref, d1_ref, wn_ref, o_ref):
    deg = d0_ref[...] + d1_ref[...]
    neigh = (p0_ref[...] + p1_ref[...]) / jnp.maximum(deg, 1.0)
    rst = s_ref[...] + jnp.dot(neigh, wn_ref[...],
                               preferred_element_type=jnp.float32)
    o_ref[...] = jnp.where(rst >= 0, rst, 0.01 * rst)

  return pl.pallas_call(
      body,
      grid=(N_NODES // BLK,),
      in_specs=[
          pl.BlockSpec((BLK, D), lambda i: (i, 0)),
          pl.BlockSpec((BLK, D), lambda i: (i, 0)),
          pl.BlockSpec((BLK, D), lambda i: (i, 0)),
          pl.BlockSpec((BLK, 1), lambda i: (i, 0)),
          pl.BlockSpec((BLK, 1), lambda i: (i, 0)),
          pl.BlockSpec((D, D), lambda i: (0, 0)),
      ],
      out_specs=pl.BlockSpec((BLK, D), lambda i: (i, 0)),
      out_shape=jax.ShapeDtypeStruct((N_NODES, D), jnp.float32),
  )(selfterm, p0, p1, d0, d1, W_neigh)


def kernel(node_embeddings, edge_index, edge_weight, W_self, W_neigh, bias):
  flat = edge_index.astype(jnp.int32).reshape(2 * N_EDGES)
  w = edge_weight.astype(jnp.float32)

  selfterm = _tc_self(node_embeddings, W_self, bias.reshape(1, D))
  p0, p1, d0, d1 = _sc_aggregate(node_embeddings, flat, flat, w)

  return _tc_combine(
      selfterm, p0, p1,
      d0.reshape(N_PAD, 1), d1.reshape(N_PAD, 1),
      W_neigh,
  )


# speedup vs baseline: 12.1601x; 1.0204x over previous
import functools

import jax
import jax.numpy as jnp
from jax import lax
from jax.experimental import pallas as pl
from jax.experimental.pallas import tpu as pltpu
from jax.experimental.pallas import tpu_sc as plsc

N_NODES = 10000
N_EDGES = 320000
D = 128

NC = 2
NS = 16
NW = NC * NS
CHUNK = 128
E_PER_W = N_EDGES // NW
K_FULL = E_PER_W // CHUNK
TAIL = E_PER_W - K_FULL * CHUNK
N_PAD = 10240
ROWS_PER_TILE = N_PAD // NS


def _sc_aggregate(x, src_idx, dst_idx, edge_weight):
  mesh = plsc.VectorSubcoreMesh(core_axis_name="c", subcore_axis_name="s")
  K = K_FULL

  @functools.partial(
      pl.kernel,
      out_type=[
          jax.ShapeDtypeStruct((N_PAD, D), jnp.float32),
          jax.ShapeDtypeStruct((N_PAD, D), jnp.float32),
          jax.ShapeDtypeStruct((N_PAD,), jnp.float32),
          jax.ShapeDtypeStruct((N_PAD,), jnp.float32),
      ],
      mesh=mesh,
      scratch_types=[
          pltpu.VMEM((2, CHUNK), jnp.int32),
          pltpu.VMEM((2, 2, CHUNK // 2), jnp.int32),
          pltpu.VMEM((2, CHUNK), jnp.float32),
          pltpu.VMEM((CHUNK,), jnp.float32),
          pltpu.VMEM((CHUNK, D), jnp.float32),
          pltpu.VMEM((CHUNK, D), jnp.float32),
          pltpu.VMEM((TAIL,), jnp.int32),
          pltpu.VMEM((TAIL,), jnp.int32),
          pltpu.VMEM((ROWS_PER_TILE,), jnp.float32),
          pltpu.VMEM_SHARED((N_PAD, D), jnp.float32),
          pltpu.VMEM_SHARED((N_PAD,), jnp.float32),
          pltpu.SemaphoreType.DMA,
          pltpu.SemaphoreType.DMA,
          pltpu.SemaphoreType.DMA,
          pltpu.SemaphoreType.DMA,
          pltpu.SemaphoreType.DMA,
          pltpu.SemaphoreType.DMA,
          pltpu.SemaphoreType.DMA,
          pltpu.SemaphoreType.DMA,
      ],
  )
  def agg(x_hbm, si_hbm, di_hbm, ew_hbm, p0_out, p1_out, d0_out, d1_out,
          sidxb, didxb, wb, ones_v, rows0, rows1, sidx_t, didx_t, zdeg,
          acc, dacc, sm0, sm1, sg0, sg1, ss0, ss1, sd0, sd1):
    cid = lax.axis_index("c")
    sid = lax.axis_index("s")
    wid = cid * NS + sid
    e_base = wid * E_PER_W
    zero16 = jnp.zeros((16,), jnp.float32)
    one16 = jnp.ones((16,), jnp.float32)
    rows = (rows0, rows1)
    sm = (sm0, sm1)
    sg = (sg0, sg1)
    ss = (ss0, ss1)
    sd = (sd0, sd1)

    H = CHUNK // 2

    def meta_descs(k, b):
      e0 = e_base + k * CHUNK
      s = pltpu.make_async_copy(si_hbm.at[pl.ds(e0, CHUNK)],
                                sidxb.at[b], sm[b])
      d0 = pltpu.make_async_copy(di_hbm.at[pl.ds(N_EDGES + e0, H)],
                                 didxb.at[b, 0], sm[b])
      d1 = pltpu.make_async_copy(di_hbm.at[pl.ds(N_EDGES + e0 + H, H)],
                                 didxb.at[b, 1], sm[b])
      w = pltpu.make_async_copy(ew_hbm.at[pl.ds(e0, CHUNK)], wb.at[b], sm[b])
      return s, d0, d1, w

    def row_scatter_desc(b, h):
      return pltpu.make_async_copy(rows[b].at[pl.ds(h * H, H)],
                                   acc.at[didxb.at[b, h]], ss[b])

    def deg_scatter_desc(b, h):
      return pltpu.make_async_copy(ones_v.at[pl.ds(0, H)],
                                   dacc.at[didxb.at[b, h]], sd[b])

    def scatter_descs(b):
      return (row_scatter_desc(b, 0), row_scatter_desc(b, 1),
              deg_scatter_desc(b, 0), deg_scatter_desc(b, 1))

    for desc in meta_descs(0, 0):
      desc.start()
    for desc in meta_descs(0, 0):
      desc.wait()
    pltpu.async_copy(x_hbm.at[sidxb.at[0]], rows0, sg0)

    for j in range(CHUNK // 16):
      ones_v[pl.ds(j * 16, 16)] = one16

    def fill_zero(r, carry):
      for c in range(D // 16):
        rows1[r, pl.ds(c * 16, 16)] = zero16
      return carry
    lax.fori_loop(0, CHUNK, fill_zero, 0)
    for j in range(ROWS_PER_TILE // 16):
      zdeg[pl.ds(j * 16, 16)] = zero16
    row0 = sid * ROWS_PER_TILE
    for j in range(ROWS_PER_TILE // CHUNK):
      pltpu.sync_copy(rows1, acc.at[pl.ds(row0 + j * CHUNK, CHUNK)])
    pltpu.sync_copy(zdeg, dacc.at[pl.ds(row0, ROWS_PER_TILE)])
    plsc.subcore_barrier()

    def scale(b, h):
      def scale_group(g, c2):
        wg = wb[b, pl.ds(g * 16, 16)]
        for j in range(16):
          w = wg[j]
          i = g * 16 + j
          for c in range(D // 16):
            rows[b][i, pl.ds(c * 16, 16)] = rows[b][i, pl.ds(c * 16, 16)] * w
        return c2
      lax.fori_loop(h * (H // 16), (h + 1) * (H // 16), scale_group, 0)

    @pl.loop(0, K, step=2)
    def pipeline(k0):
      for b in (0, 1):
        k = k0 + b
        nb = 1 - b
        if b == 0:
          @pl.when(k0 > 0)
          def _():
            for desc in scatter_descs(1):
              desc.wait()
        else:
          for desc in scatter_descs(0):
            desc.wait()
        if b == 0:
          for desc in meta_descs(k + 1, 1):
            desc.start()
        else:
          @pl.when(k0 + 2 < K)
          def _():
            for desc in meta_descs(k + 1, 0):
              desc.start()
        deg_scatter_desc(b, 0).start(add=True)
        deg_scatter_desc(b, 1).start(add=True)
        pltpu.make_async_copy(x_hbm.at[sidxb.at[b]], rows[b], sg[b]).wait()
        if b == 0:
          for desc in meta_descs(k + 1, 1):
            desc.wait()
          pltpu.async_copy(x_hbm.at[sidxb.at[1]], rows[1], sg[1])
        else:
          @pl.when(k0 + 2 < K)
          def _():
            for desc in meta_descs(k + 1, 0):
              desc.wait()
            pltpu.async_copy(x_hbm.at[sidxb.at[0]], rows[0], sg[0])
        scale(b, 0)
        row_scatter_desc(b, 0).start(add=True)
        scale(b, 1)
        row_scatter_desc(b, 1).start(add=True)

    for desc in scatter_descs(1):
      desc.wait()

    e0 = e_base + K * CHUNK
    pltpu.sync_copy(si_hbm.at[pl.ds(e0, TAIL)], sidx_t)
    pltpu.sync_copy(di_hbm.at[pl.ds(N_EDGES + e0, TAIL)], didx_t)
    pltpu.sync_copy(ew_hbm.at[pl.ds(e0, TAIL)], wb.at[0, pl.ds(0, TAIL)])
    pltpu.async_copy(x_hbm.at[sidx_t], rows0.at[pl.ds(0, TAIL)], sg0).wait()
    wg = wb[0, pl.ds(0, 16)]
    for j in range(TAIL):
      w = wg[j]
      for c in range(D // 16):
        rows0[j, pl.ds(c * 16, 16)] = rows0[j, pl.ds(c * 16, 16)] * w
    pltpu.sync_copy(rows0.at[pl.ds(0, TAIL)], acc.at[didx_t], add=True)
    pltpu.sync_copy(ones_v.at[pl.ds(0, TAIL)], dacc.at[didx_t], add=True)
    plsc.subcore_barrier()

    @pl.when(cid == 0)
    def _():
      pltpu.sync_copy(acc.at[pl.ds(row0, ROWS_PER_TILE)],
                      p0_out.at[pl.ds(row0, ROWS_PER_TILE)])
      pltpu.sync_copy(dacc.at[pl.ds(row0, ROWS_PER_TILE)],
                      d0_out.at[pl.ds(row0, ROWS_PER_TILE)])

    @pl.when(cid == 1)
    def _():
      pltpu.sync_copy(acc.at[pl.ds(row0, ROWS_PER_TILE)],
                      p1_out.at[pl.ds(row0, ROWS_PER_TILE)])
      pltpu.sync_copy(dacc.at[pl.ds(row0, ROWS_PER_TILE)],
                      d1_out.at[pl.ds(row0, ROWS_PER_TILE)])

  return agg(x, src_idx, dst_idx, edge_weight)


BLK = 2000


def _tc_self(x, W_self, bias):
  def body(x_ref, ws_ref, b_ref, o_ref):
    o_ref[...] = (
        jnp.dot(x_ref[...], ws_ref[...], preferred_element_type=jnp.float32)
        + b_ref[...]
    )

  return pl.pallas_call(
      body,
      grid=(N_NODES // BLK,),
      in_specs=[
          pl.BlockSpec((BLK, D), lambda i: (i, 0)),
          pl.BlockSpec((D, D), lambda i: (0, 0)),
          pl.BlockSpec((1, D), lambda i: (0, 0)),
      ],
      out_specs=pl.BlockSpec((BLK, D), lambda i: (i, 0)),
      out_shape=jax.ShapeDtypeStruct((N_NODES, D), jnp.float32),
  )(x, W_self, bias)


def _tc_combine(selfterm, p0, p1, d0, d1, W_neigh):
  def body(s_ref, p0_ref, p1_ref, d0_ref, d1_ref, wn_ref, o_ref):
    deg = d0_ref[...] + d1_ref[...]
    neigh = (p0_ref[...] + p1_ref[...]) / jnp.maximum(deg, 1.0)
    rst = s_ref[...] + jnp.dot(neigh, wn_ref[...],
                               preferred_element_type=jnp.float32)
    o_ref[...] = jnp.where(rst >= 0, rst, 0.01 * rst)

  return pl.pallas_call(
      body,
      grid=(N_NODES // BLK,),
      in_specs=[
          pl.BlockSpec((BLK, D), lambda i: (i, 0)),
          pl.BlockSpec((BLK, D), lambda i: (i, 0)),
          pl.BlockSpec((BLK, D), lambda i: (i, 0)),
          pl.BlockSpec((BLK, 1), lambda i: (i, 0)),
          pl.BlockSpec((BLK, 1), lambda i: (i, 0)),
          pl.BlockSpec((D, D), lambda i: (0, 0)),
      ],
      out_specs=pl.BlockSpec((BLK, D), lambda i: (i, 0)),
      out_shape=jax.ShapeDtypeStruct((N_NODES, D), jnp.float32),
  )(selfterm, p0, p1, d0, d1, W_neigh)


def kernel(node_embeddings, edge_index, edge_weight, W_self, W_neigh, bias):
  flat = edge_index.astype(jnp.int32).reshape(2 * N_EDGES)
  w = edge_weight.astype(jnp.float32)

  selfterm = _tc_self(node_embeddings, W_self, bias.reshape(1, D))
  p0, p1, d0, d1 = _sc_aggregate(node_embeddings, flat, flat, w)

  return _tc_combine(
      selfterm, p0, p1,
      d0.reshape(N_PAD, 1), d1.reshape(N_PAD, 1),
      W_neigh,
  )


# speedup vs baseline: 12.1821x; 1.0018x over previous
import functools

import jax
import jax.numpy as jnp
from jax import lax
from jax.experimental import pallas as pl
from jax.experimental.pallas import tpu as pltpu
from jax.experimental.pallas import tpu_sc as plsc

N_NODES = 10000
N_EDGES = 320000
D = 128

NC = 2
NS = 16
NW = NC * NS
CHUNK = 128
E_PER_W = N_EDGES // NW
K_FULL = E_PER_W // CHUNK
TAIL = E_PER_W - K_FULL * CHUNK
N_PAD = 10240
ROWS_PER_TILE = N_PAD // NS


def _sc_aggregate(x, src_idx, dst_idx, edge_weight):
  mesh = plsc.VectorSubcoreMesh(core_axis_name="c", subcore_axis_name="s")
  K = K_FULL

  @functools.partial(
      pl.kernel,
      out_type=[
          jax.ShapeDtypeStruct((N_PAD, D), jnp.float32),
          jax.ShapeDtypeStruct((N_PAD, D), jnp.float32),
          jax.ShapeDtypeStruct((N_PAD,), jnp.float32),
          jax.ShapeDtypeStruct((N_PAD,), jnp.float32),
      ],
      mesh=mesh,
      scratch_types=[
          pltpu.VMEM((2, CHUNK), jnp.int32),
          pltpu.VMEM((2, 2, CHUNK // 2), jnp.int32),
          pltpu.VMEM((2, CHUNK), jnp.float32),
          pltpu.VMEM((CHUNK,), jnp.float32),
          pltpu.VMEM((CHUNK, D), jnp.float32),
          pltpu.VMEM((CHUNK, D), jnp.float32),
          pltpu.VMEM((TAIL,), jnp.int32),
          pltpu.VMEM((TAIL,), jnp.int32),
          pltpu.VMEM((ROWS_PER_TILE,), jnp.float32),
          pltpu.VMEM_SHARED((N_PAD, D), jnp.float32),
          pltpu.VMEM_SHARED((N_PAD,), jnp.float32),
          pltpu.SemaphoreType.DMA,
          pltpu.SemaphoreType.DMA,
          pltpu.SemaphoreType.DMA,
          pltpu.SemaphoreType.DMA,
          pltpu.SemaphoreType.DMA,
          pltpu.SemaphoreType.DMA,
          pltpu.SemaphoreType.DMA,
          pltpu.SemaphoreType.DMA,
      ],
  )
  def agg(x_hbm, si_hbm, di_hbm, ew_hbm, p0_out, p1_out, d0_out, d1_out,
          sidxb, didxb, wb, ones_v, rows0, rows1, sidx_t, didx_t, zdeg,
          acc, dacc, sm0, sm1, sg0, sg1, ss0, ss1, sd0, sd1):
    cid = lax.axis_index("c")
    sid = lax.axis_index("s")
    wid = cid * NS + sid
    e_base = wid * E_PER_W
    zero16 = jnp.zeros((16,), jnp.float32)
    one16 = jnp.ones((16,), jnp.float32)
    rows = (rows0, rows1)
    sm = (sm0, sm1)
    sg = (sg0, sg1)
    ss = (ss0, ss1)
    sd = (sd0, sd1)

    H = CHUNK // 2

    def meta_descs(k, b):
      e0 = e_base + k * CHUNK
      s = pltpu.make_async_copy(si_hbm.at[pl.ds(e0, CHUNK)],
                                sidxb.at[b], sm[b])
      d0 = pltpu.make_async_copy(di_hbm.at[pl.ds(N_EDGES + e0, H)],
                                 didxb.at[b, 0], sm[b])
      d1 = pltpu.make_async_copy(di_hbm.at[pl.ds(N_EDGES + e0 + H, H)],
                                 didxb.at[b, 1], sm[b])
      w = pltpu.make_async_copy(ew_hbm.at[pl.ds(e0, CHUNK)], wb.at[b], sm[b])
      return s, d0, d1, w

    def row_scatter_desc(b, h):
      return pltpu.make_async_copy(rows[b].at[pl.ds(h * H, H)],
                                   acc.at[didxb.at[b, h]], ss[b])

    def deg_scatter_desc(b, h):
      return pltpu.make_async_copy(ones_v.at[pl.ds(0, H)],
                                   dacc.at[didxb.at[b, h]], sd[b])

    def scatter_descs(b):
      return (row_scatter_desc(b, 0), row_scatter_desc(b, 1),
              deg_scatter_desc(b, 0), deg_scatter_desc(b, 1))

    for desc in meta_descs(0, 0):
      desc.start()
    for desc in meta_descs(0, 0):
      desc.wait()
    pltpu.async_copy(x_hbm.at[sidxb.at[0]], rows0, sg0)

    for j in range(CHUNK // 16):
      ones_v[pl.ds(j * 16, 16)] = one16

    def fill_zero(r, carry):
      for c in range(D // 16):
        rows1[r, pl.ds(c * 16, 16)] = zero16
      return carry
    lax.fori_loop(0, CHUNK, fill_zero, 0)
    for j in range(ROWS_PER_TILE // 16):
      zdeg[pl.ds(j * 16, 16)] = zero16
    row0 = sid * ROWS_PER_TILE
    for j in range(ROWS_PER_TILE // CHUNK):
      pltpu.sync_copy(rows1, acc.at[pl.ds(row0 + j * CHUNK, CHUNK)])
    pltpu.sync_copy(zdeg, dacc.at[pl.ds(row0, ROWS_PER_TILE)])
    plsc.subcore_barrier()

    def scale(b, h):
      def scale_group(g, c2):
        wg = wb[b, pl.ds(g * 16, 16)]
        for j in range(16):
          w = wg[j]
          i = g * 16 + j
          for c in range(D // 16):
            rows[b][i, pl.ds(c * 16, 16)] = rows[b][i, pl.ds(c * 16, 16)] * w
        return c2
      lax.fori_loop(h * (H // 16), (h + 1) * (H // 16), scale_group, 0)

    @pl.loop(0, K, step=2)
    def pipeline(k0):
      for b in (0, 1):
        k = k0 + b
        nb = 1 - b
        if b == 0:
          @pl.when(k0 > 0)
          def _():
            for desc in scatter_descs(1):
              desc.wait()
        else:
          for desc in scatter_descs(0):
            desc.wait()
        if b == 0:
          for desc in meta_descs(k + 1, 1):
            desc.start()
        else:
          @pl.when(k0 + 2 < K)
          def _():
            for desc in meta_descs(k + 1, 0):
              desc.start()
        deg_scatter_desc(b, 0).start(add=True)
        deg_scatter_desc(b, 1).start(add=True)
        pltpu.make_async_copy(x_hbm.at[sidxb.at[b]], rows[b], sg[b]).wait()
        if b == 0:
          for desc in meta_descs(k + 1, 1):
            desc.wait()
          pltpu.async_copy(x_hbm.at[sidxb.at[1]], rows[1], sg[1])
        else:
          @pl.when(k0 + 2 < K)
          def _():
            for desc in meta_descs(k + 1, 0):
              desc.wait()
            pltpu.async_copy(x_hbm.at[sidxb.at[0]], rows[0], sg[0])
        scale(b, 0)
        row_scatter_desc(b, 0).start(add=True)
        scale(b, 1)
        row_scatter_desc(b, 1).start(add=True)

    for desc in scatter_descs(1):
      desc.wait()

    e0 = e_base + K * CHUNK
    pltpu.sync_copy(si_hbm.at[pl.ds(e0, TAIL)], sidx_t)
    pltpu.sync_copy(di_hbm.at[pl.ds(N_EDGES + e0, TAIL)], didx_t)
    pltpu.sync_copy(ew_hbm.at[pl.ds(e0, TAIL)], wb.at[0, pl.ds(0, TAIL)])
    pltpu.async_copy(x_hbm.at[sidx_t], rows0.at[pl.ds(0, TAIL)], sg0).wait()
    wg = wb[0, pl.ds(0, 16)]
    for j in range(TAIL):
      w = wg[j]
      for c in range(D // 16):
        rows0[j, pl.ds(c * 16, 16)] = rows0[j, pl.ds(c * 16, 16)] * w
    pltpu.sync_copy(rows0.at[pl.ds(0, TAIL)], acc.at[didx_t], add=True)
    pltpu.sync_copy(ones_v.at[pl.ds(0, TAIL)], dacc.at[didx_t], add=True)
    plsc.subcore_barrier()

    @pl.when(cid == 0)
    def _():
      pltpu.sync_copy(acc.at[pl.ds(row0, ROWS_PER_TILE)],
                      p0_out.at[pl.ds(row0, ROWS_PER_TILE)])
      pltpu.sync_copy(dacc.at[pl.ds(row0, ROWS_PER_TILE)],
                      d0_out.at[pl.ds(row0, ROWS_PER_TILE)])

    @pl.when(cid == 1)
    def _():
      pltpu.sync_copy(acc.at[pl.ds(row0, ROWS_PER_TILE)],
                      p1_out.at[pl.ds(row0, ROWS_PER_TILE)])
      pltpu.sync_copy(dacc.at[pl.ds(row0, ROWS_PER_TILE)],
                      d1_out.at[pl.ds(row0, ROWS_PER_TILE)])

  return agg(x, src_idx, dst_idx, edge_weight)


BLK = 2000


def _tc_combine(x, p0, p1, d0, d1, W_self, W_neigh, bias):
  def body(x_ref, p0_ref, p1_ref, d0_ref, d1_ref, ws_ref, wn_ref, b_ref,
           o_ref):
    deg = d0_ref[...] + d1_ref[...]
    neigh = (p0_ref[...] + p1_ref[...]) / jnp.maximum(deg, 1.0)
    rst = (
        jnp.dot(x_ref[...], ws_ref[...], preferred_element_type=jnp.float32)
        + jnp.dot(neigh, wn_ref[...], preferred_element_type=jnp.float32)
        + b_ref[...]
    )
    o_ref[...] = jnp.where(rst >= 0, rst, 0.01 * rst)

  return pl.pallas_call(
      body,
      grid=(N_NODES // BLK,),
      in_specs=[
          pl.BlockSpec((BLK, D), lambda i: (i, 0)),
          pl.BlockSpec((BLK, D), lambda i: (i, 0)),
          pl.BlockSpec((BLK, D), lambda i: (i, 0)),
          pl.BlockSpec((BLK, 1), lambda i: (i, 0)),
          pl.BlockSpec((BLK, 1), lambda i: (i, 0)),
          pl.BlockSpec((D, D), lambda i: (0, 0)),
          pl.BlockSpec((D, D), lambda i: (0, 0)),
          pl.BlockSpec((1, D), lambda i: (0, 0)),
      ],
      out_specs=pl.BlockSpec((BLK, D), lambda i: (i, 0)),
      out_shape=jax.ShapeDtypeStruct((N_NODES, D), jnp.float32),
  )(x, p0, p1, d0, d1, W_self, W_neigh, bias)


def kernel(node_embeddings, edge_index, edge_weight, W_self, W_neigh, bias):
  flat = edge_index.astype(jnp.int32).reshape(2 * N_EDGES)
  w = edge_weight.astype(jnp.float32)

  p0, p1, d0, d1 = _sc_aggregate(node_embeddings, flat, flat, w)

  return _tc_combine(
      node_embeddings, p0, p1,
      d0.reshape(N_PAD, 1), d1.reshape(N_PAD, 1),
      W_self, W_neigh, bias.reshape(1, D),
  )


# speedup vs baseline: 12.3180x; 1.0112x over previous
import functools

import jax
import jax.numpy as jnp
from jax import lax
from jax.experimental import pallas as pl
from jax.experimental.pallas import tpu as pltpu
from jax.experimental.pallas import tpu_sc as plsc

N_NODES = 10000
N_EDGES = 320000
D = 128

NC = 2
NS = 16
NW = NC * NS
CHUNK = 128
E_PER_W = N_EDGES // NW
K_FULL = E_PER_W // CHUNK
TAIL = E_PER_W - K_FULL * CHUNK
N_PAD = 10240
ROWS_PER_TILE = N_PAD // NS


def _sc_aggregate(x, src_idx, dst_idx, edge_weight):
  mesh = plsc.VectorSubcoreMesh(core_axis_name="c", subcore_axis_name="s")
  K = K_FULL

  @functools.partial(
      pl.kernel,
      out_type=[
          jax.ShapeDtypeStruct((N_PAD, D), jnp.float32),
          jax.ShapeDtypeStruct((N_PAD, D), jnp.float32),
          jax.ShapeDtypeStruct((N_PAD,), jnp.float32),
          jax.ShapeDtypeStruct((N_PAD,), jnp.float32),
      ],
      mesh=mesh,
      scratch_types=[
          pltpu.VMEM((2, CHUNK), jnp.int32),
          pltpu.VMEM((2, 2, CHUNK // 2), jnp.int32),
          pltpu.VMEM((2, CHUNK), jnp.float32),
          pltpu.VMEM((CHUNK,), jnp.float32),
          pltpu.VMEM((CHUNK, D), jnp.float32),
          pltpu.VMEM((CHUNK, D), jnp.float32),
          pltpu.VMEM((TAIL,), jnp.int32),
          pltpu.VMEM((TAIL,), jnp.int32),
          pltpu.VMEM((TAIL,), jnp.float32),
          pltpu.VMEM((TAIL, D), jnp.float32),
          pltpu.VMEM((ROWS_PER_TILE,), jnp.float32),
          pltpu.VMEM_SHARED((N_PAD, D), jnp.float32),
          pltpu.VMEM_SHARED((N_PAD,), jnp.float32),
          pltpu.SemaphoreType.DMA,
          pltpu.SemaphoreType.DMA,
          pltpu.SemaphoreType.DMA,
          pltpu.SemaphoreType.DMA,
          pltpu.SemaphoreType.DMA,
          pltpu.SemaphoreType.DMA,
          pltpu.SemaphoreType.DMA,
          pltpu.SemaphoreType.DMA,
          pltpu.SemaphoreType.DMA,
          pltpu.SemaphoreType.DMA,
      ],
  )
  def agg(x_hbm, si_hbm, di_hbm, ew_hbm, p0_out, p1_out, d0_out, d1_out,
          sidxb, didxb, wb, ones_v, rows0, rows1, sidx_t, didx_t, wt,
          rows_t, zdeg, acc, dacc, sm0, sm1, sg0, sg1, ss0, ss1, sd0, sd1,
          st_m, st_s):
    cid = lax.axis_index("c")
    sid = lax.axis_index("s")
    wid = cid * NS + sid
    e_base = wid * E_PER_W
    zero16 = jnp.zeros((16,), jnp.float32)
    one16 = jnp.ones((16,), jnp.float32)
    rows = (rows0, rows1)
    sm = (sm0, sm1)
    sg = (sg0, sg1)
    ss = (ss0, ss1)
    sd = (sd0, sd1)

    H = CHUNK // 2

    def meta_descs(k, b):
      e0 = e_base + k * CHUNK
      s = pltpu.make_async_copy(si_hbm.at[pl.ds(e0, CHUNK)],
                                sidxb.at[b], sm[b])
      d0 = pltpu.make_async_copy(di_hbm.at[pl.ds(N_EDGES + e0, H)],
                                 didxb.at[b, 0], sm[b])
      d1 = pltpu.make_async_copy(di_hbm.at[pl.ds(N_EDGES + e0 + H, H)],
                                 didxb.at[b, 1], sm[b])
      w = pltpu.make_async_copy(ew_hbm.at[pl.ds(e0, CHUNK)], wb.at[b], sm[b])
      return s, d0, d1, w

    def row_scatter_desc(b, h):
      return pltpu.make_async_copy(rows[b].at[pl.ds(h * H, H)],
                                   acc.at[didxb.at[b, h]], ss[b])

    def deg_scatter_desc(b, h):
      return pltpu.make_async_copy(ones_v.at[pl.ds(0, H)],
                                   dacc.at[didxb.at[b, h]], sd[b])

    def scatter_descs(b):
      return (row_scatter_desc(b, 0), row_scatter_desc(b, 1),
              deg_scatter_desc(b, 0), deg_scatter_desc(b, 1))

    e0t = e_base + K * CHUNK
    tail_meta = (
        pltpu.make_async_copy(si_hbm.at[pl.ds(e0t, TAIL)], sidx_t, st_m),
        pltpu.make_async_copy(di_hbm.at[pl.ds(N_EDGES + e0t, TAIL)],
                              didx_t, st_m),
        pltpu.make_async_copy(ew_hbm.at[pl.ds(e0t, TAIL)], wt, st_m),
    )
    for desc in meta_descs(0, 0) + tail_meta:
      desc.start()
    for desc in meta_descs(0, 0) + tail_meta:
      desc.wait()
    pltpu.async_copy(x_hbm.at[sidxb.at[0]], rows0, sg0)
    tail_gather = pltpu.make_async_copy(x_hbm.at[sidx_t], rows_t, st_m)
    tail_gather.start()

    for j in range(CHUNK // 16):
      ones_v[pl.ds(j * 16, 16)] = one16

    def fill_zero(r, carry):
      for c in range(D // 16):
        rows1[r, pl.ds(c * 16, 16)] = zero16
      return carry
    lax.fori_loop(0, CHUNK, fill_zero, 0)
    for j in range(ROWS_PER_TILE // 16):
      zdeg[pl.ds(j * 16, 16)] = zero16
    row0 = sid * ROWS_PER_TILE
    for j in range(ROWS_PER_TILE // CHUNK):
      pltpu.sync_copy(rows1, acc.at[pl.ds(row0 + j * CHUNK, CHUNK)])
    pltpu.sync_copy(zdeg, dacc.at[pl.ds(row0, ROWS_PER_TILE)])
    plsc.subcore_barrier()

    tail_gather.wait()
    for g in range(TAIL // 16):
      wg = wt[pl.ds(g * 16, 16)]
      for j in range(16):
        w = wg[j]
        i = g * 16 + j
        for c in range(D // 16):
          rows_t[i, pl.ds(c * 16, 16)] = rows_t[i, pl.ds(c * 16, 16)] * w
    tail_scatters = (
        pltpu.make_async_copy(rows_t, acc.at[didx_t], st_s),
        pltpu.make_async_copy(ones_v.at[pl.ds(0, TAIL)], dacc.at[didx_t],
                              st_s),
    )
    for desc in tail_scatters:
      desc.start(add=True)

    def scale(b, h):
      def scale_group(g, c2):
        wg = wb[b, pl.ds(g * 16, 16)]
        for j in range(16):
          w = wg[j]
          i = g * 16 + j
          for c in range(D // 16):
            rows[b][i, pl.ds(c * 16, 16)] = rows[b][i, pl.ds(c * 16, 16)] * w
        return c2
      lax.fori_loop(h * (H // 16), (h + 1) * (H // 16), scale_group, 0)

    @pl.loop(0, K, step=2)
    def pipeline(k0):
      for b in (0, 1):
        k = k0 + b
        nb = 1 - b
        if b == 0:
          @pl.when(k0 > 0)
          def _():
            for desc in scatter_descs(1):
              desc.wait()
        else:
          for desc in scatter_descs(0):
            desc.wait()
        if b == 0:
          for desc in meta_descs(k + 1, 1):
            desc.start()
        else:
          @pl.when(k0 + 2 < K)
          def _():
            for desc in meta_descs(k + 1, 0):
              desc.start()
        deg_scatter_desc(b, 0).start(add=True)
        deg_scatter_desc(b, 1).start(add=True)
        pltpu.make_async_copy(x_hbm.at[sidxb.at[b]], rows[b], sg[b]).wait()
        if b == 0:
          for desc in meta_descs(k + 1, 1):
            desc.wait()
          pltpu.async_copy(x_hbm.at[sidxb.at[1]], rows[1], sg[1])
        else:
          @pl.when(k0 + 2 < K)
          def _():
            for desc in meta_descs(k + 1, 0):
              desc.wait()
            pltpu.async_copy(x_hbm.at[sidxb.at[0]], rows[0], sg[0])
        scale(b, 0)
        row_scatter_desc(b, 0).start(add=True)
        scale(b, 1)
        row_scatter_desc(b, 1).start(add=True)

    for desc in scatter_descs(1):
      desc.wait()
    for desc in tail_scatters:
      desc.wait()
    plsc.subcore_barrier()

    @pl.when(cid == 0)
    def _():
      pltpu.sync_copy(acc.at[pl.ds(row0, ROWS_PER_TILE)],
                      p0_out.at[pl.ds(row0, ROWS_PER_TILE)])
      pltpu.sync_copy(dacc.at[pl.ds(row0, ROWS_PER_TILE)],
                      d0_out.at[pl.ds(row0, ROWS_PER_TILE)])

    @pl.when(cid == 1)
    def _():
      pltpu.sync_copy(acc.at[pl.ds(row0, ROWS_PER_TILE)],
                      p1_out.at[pl.ds(row0, ROWS_PER_TILE)])
      pltpu.sync_copy(dacc.at[pl.ds(row0, ROWS_PER_TILE)],
                      d1_out.at[pl.ds(row0, ROWS_PER_TILE)])

  return agg(x, src_idx, dst_idx, edge_weight)


BLK = 2000


def _tc_combine(x, p0, p1, d0, d1, W_self, W_neigh, bias):
  def body(x_ref, p0_ref, p1_ref, d0_ref, d1_ref, ws_ref, wn_ref, b_ref,
           o_ref):
    deg = d0_ref[...] + d1_ref[...]
    neigh = (p0_ref[...] + p1_ref[...]) / jnp.maximum(deg, 1.0)
    rst = (
        jnp.dot(x_ref[...], ws_ref[...], preferred_element_type=jnp.float32)
        + jnp.dot(neigh, wn_ref[...], preferred_element_type=jnp.float32)
        + b_ref[...]
    )
    o_ref[...] = jnp.where(rst >= 0, rst, 0.01 * rst)

  return pl.pallas_call(
      body,
      grid=(N_NODES // BLK,),
      in_specs=[
          pl.BlockSpec((BLK, D), lambda i: (i, 0)),
          pl.BlockSpec((BLK, D), lambda i: (i, 0)),
          pl.BlockSpec((BLK, D), lambda i: (i, 0)),
          pl.BlockSpec((BLK, 1), lambda i: (i, 0)),
          pl.BlockSpec((BLK, 1), lambda i: (i, 0)),
          pl.BlockSpec((D, D), lambda i: (0, 0)),
          pl.BlockSpec((D, D), lambda i: (0, 0)),
          pl.BlockSpec((1, D), lambda i: (0, 0)),
      ],
      out_specs=pl.BlockSpec((BLK, D), lambda i: (i, 0)),
      out_shape=jax.ShapeDtypeStruct((N_NODES, D), jnp.float32),
  )(x, p0, p1, d0, d1, W_self, W_neigh, bias)


def kernel(node_embeddings, edge_index, edge_weight, W_self, W_neigh, bias):
  flat = edge_index.astype(jnp.int32).reshape(2 * N_EDGES)
  w = edge_weight.astype(jnp.float32)

  p0, p1, d0, d1 = _sc_aggregate(node_embeddings, flat, flat, w)

  return _tc_combine(
      node_embeddings, p0, p1,
      d0.reshape(N_PAD, 1), d1.reshape(N_PAD, 1),
      W_self, W_neigh, bias.reshape(1, D),
  )


# speedup vs baseline: 12.3829x; 1.0053x over previous
import functools

import jax
import jax.numpy as jnp
from jax import lax
from jax.experimental import pallas as pl
from jax.experimental.pallas import tpu as pltpu
from jax.experimental.pallas import tpu_sc as plsc

N_NODES = 10000
N_EDGES = 320000
D = 128

NC = 2
NS = 16
NW = NC * NS
CHUNK = 128
E_PER_W = N_EDGES // NW
K_FULL = E_PER_W // CHUNK
TAIL = E_PER_W - K_FULL * CHUNK
N_PAD = 10240
ROWS_PER_TILE = N_PAD // NS


def _sc_aggregate(x, src_idx, dst_idx, edge_weight):
  mesh = plsc.VectorSubcoreMesh(core_axis_name="c", subcore_axis_name="s")
  K = K_FULL

  @functools.partial(
      pl.kernel,
      out_type=[
          jax.ShapeDtypeStruct((N_PAD, D), jnp.float32),
          jax.ShapeDtypeStruct((N_PAD, D), jnp.float32),
          jax.ShapeDtypeStruct((N_PAD,), jnp.float32),
          jax.ShapeDtypeStruct((N_PAD,), jnp.float32),
      ],
      mesh=mesh,
      scratch_types=[
          pltpu.VMEM((2, CHUNK), jnp.int32),
          pltpu.VMEM((2, 2, CHUNK // 2), jnp.int32),
          pltpu.VMEM((2, CHUNK), jnp.float32),
          pltpu.VMEM((CHUNK,), jnp.float32),
          pltpu.VMEM((CHUNK, D), jnp.float32),
          pltpu.VMEM((CHUNK, D), jnp.float32),
          pltpu.VMEM((TAIL,), jnp.int32),
          pltpu.VMEM((TAIL,), jnp.int32),
          pltpu.VMEM((TAIL,), jnp.float32),
          pltpu.VMEM((TAIL, D), jnp.float32),
          pltpu.VMEM((ROWS_PER_TILE,), jnp.float32),
          pltpu.VMEM_SHARED((N_PAD, D), jnp.float32),
          pltpu.VMEM_SHARED((N_PAD,), jnp.float32),
          pltpu.SemaphoreType.DMA,
          pltpu.SemaphoreType.DMA,
          pltpu.SemaphoreType.DMA,
          pltpu.SemaphoreType.DMA,
          pltpu.SemaphoreType.DMA,
          pltpu.SemaphoreType.DMA,
          pltpu.SemaphoreType.DMA,
          pltpu.SemaphoreType.DMA,
          pltpu.SemaphoreType.DMA,
          pltpu.SemaphoreType.DMA,
      ],
  )
  def agg(x_hbm, si_hbm, di_hbm, ew_hbm, p0_out, p1_out, d0_out, d1_out,
          sidxb, didxb, wb, ones_v, rows0, rows1, sidx_t, didx_t, wt,
          rows_t, zdeg, acc, dacc, sm0, sm1, sg0, sg1, ss0, ss1, sd0, sd1,
          st_m, st_s):
    cid = lax.axis_index("c")
    sid = lax.axis_index("s")
    wid = cid * NS + sid
    e_base = wid * E_PER_W
    zero16 = jnp.zeros((16,), jnp.float32)
    one16 = jnp.ones((16,), jnp.float32)
    rows = (rows0, rows1)
    sm = (sm0, sm1)
    sg = (sg0, sg1)
    ss = (ss0, ss1)
    sd = (sd0, sd1)

    H = CHUNK // 2

    def meta_descs(k, b):
      e0 = e_base + k * CHUNK
      s = pltpu.make_async_copy(si_hbm.at[pl.ds(e0, CHUNK)],
                                sidxb.at[b], sm[b])
      d0 = pltpu.make_async_copy(di_hbm.at[pl.ds(N_EDGES + e0, H)],
                                 didxb.at[b, 0], sm[b])
      d1 = pltpu.make_async_copy(di_hbm.at[pl.ds(N_EDGES + e0 + H, H)],
                                 didxb.at[b, 1], sm[b])
      w = pltpu.make_async_copy(ew_hbm.at[pl.ds(e0, CHUNK)], wb.at[b], sm[b])
      return s, d0, d1, w

    def row_scatter_desc(b, h):
      return pltpu.make_async_copy(rows[b].at[pl.ds(h * H, H)],
                                   acc.at[didxb.at[b, h]], ss[b])

    def deg_scatter_desc(b, h):
      return pltpu.make_async_copy(ones_v.at[pl.ds(0, H)],
                                   dacc.at[didxb.at[b, h]], sd[b])

    def scatter_descs(b):
      return (row_scatter_desc(b, 0), row_scatter_desc(b, 1),
              deg_scatter_desc(b, 0), deg_scatter_desc(b, 1))

    e0t = e_base + K * CHUNK
    tail_meta = (
        pltpu.make_async_copy(si_hbm.at[pl.ds(e0t, TAIL)], sidx_t, st_m),
        pltpu.make_async_copy(di_hbm.at[pl.ds(N_EDGES + e0t, TAIL)],
                              didx_t, st_m),
        pltpu.make_async_copy(ew_hbm.at[pl.ds(e0t, TAIL)], wt, st_m),
    )
    for desc in meta_descs(0, 0) + tail_meta:
      desc.start()
    for desc in meta_descs(0, 0) + tail_meta:
      desc.wait()
    pltpu.async_copy(x_hbm.at[sidxb.at[0]], rows0, sg0)
    tail_gather = pltpu.make_async_copy(x_hbm.at[sidx_t], rows_t, st_m)
    tail_gather.start()

    for j in range(CHUNK // 16):
      ones_v[pl.ds(j * 16, 16)] = one16

    def fill_zero(r, carry):
      for c in range(D // 16):
        rows1[r, pl.ds(c * 16, 16)] = zero16
      return carry
    lax.fori_loop(0, CHUNK, fill_zero, 0)
    for j in range(ROWS_PER_TILE // 16):
      zdeg[pl.ds(j * 16, 16)] = zero16
    row0 = sid * ROWS_PER_TILE
    zero_descs = tuple(
        pltpu.make_async_copy(rows1, acc.at[pl.ds(row0 + j * CHUNK, CHUNK)],
                              st_s)
        for j in range(ROWS_PER_TILE // CHUNK)
    ) + (pltpu.make_async_copy(zdeg, dacc.at[pl.ds(row0, ROWS_PER_TILE)],
                               st_s),)
    for desc in zero_descs:
      desc.start()
    for desc in zero_descs:
      desc.wait()
    plsc.subcore_barrier()

    tail_gather.wait()
    for g in range(TAIL // 16):
      wg = wt[pl.ds(g * 16, 16)]
      for j in range(16):
        w = wg[j]
        i = g * 16 + j
        for c in range(D // 16):
          rows_t[i, pl.ds(c * 16, 16)] = rows_t[i, pl.ds(c * 16, 16)] * w
    tail_scatters = (
        pltpu.make_async_copy(rows_t, acc.at[didx_t], st_s),
        pltpu.make_async_copy(ones_v.at[pl.ds(0, TAIL)], dacc.at[didx_t],
                              st_s),
    )
    for desc in tail_scatters:
      desc.start(add=True)

    def scale(b, h):
      def scale_group(g, c2):
        wg = wb[b, pl.ds(g * 16, 16)]
        for j in range(16):
          w = wg[j]
          i = g * 16 + j
          for c in range(D // 16):
            rows[b][i, pl.ds(c * 16, 16)] = rows[b][i, pl.ds(c * 16, 16)] * w
        return c2
      lax.fori_loop(h * (H // 16), (h + 1) * (H // 16), scale_group, 0)

    @pl.loop(0, K, step=2)
    def pipeline(k0):
      for b in (0, 1):
        k = k0 + b
        nb = 1 - b
        if b == 0:
          @pl.when(k0 > 0)
          def _():
            for desc in scatter_descs(1):
              desc.wait()
        else:
          for desc in scatter_descs(0):
            desc.wait()
        if b == 0:
          for desc in meta_descs(k + 1, 1):
            desc.start()
        else:
          @pl.when(k0 + 2 < K)
          def _():
            for desc in meta_descs(k + 1, 0):
              desc.start()
        deg_scatter_desc(b, 0).start(add=True)
        deg_scatter_desc(b, 1).start(add=True)
        pltpu.make_async_copy(x_hbm.at[sidxb.at[b]], rows[b], sg[b]).wait()
        if b == 0:
          for desc in meta_descs(k + 1, 1):
            desc.wait()
          pltpu.async_copy(x_hbm.at[sidxb.at[1]], rows[1], sg[1])
        else:
          @pl.when(k0 + 2 < K)
          def _():
            for desc in meta_descs(k + 1, 0):
              desc.wait()
            pltpu.async_copy(x_hbm.at[sidxb.at[0]], rows[0], sg[0])
        scale(b, 0)
        row_scatter_desc(b, 0).start(add=True)
        scale(b, 1)
        row_scatter_desc(b, 1).start(add=True)

    for desc in scatter_descs(1):
      desc.wait()
    for desc in tail_scatters:
      desc.wait()
    plsc.subcore_barrier()

    @pl.when(cid == 0)
    def _():
      a = pltpu.make_async_copy(acc.at[pl.ds(row0, ROWS_PER_TILE)],
                                p0_out.at[pl.ds(row0, ROWS_PER_TILE)], st_s)
      b = pltpu.make_async_copy(dacc.at[pl.ds(row0, ROWS_PER_TILE)],
                                d0_out.at[pl.ds(row0, ROWS_PER_TILE)], st_m)
      a.start()
      b.start()
      a.wait()
      b.wait()

    @pl.when(cid == 1)
    def _():
      a = pltpu.make_async_copy(acc.at[pl.ds(row0, ROWS_PER_TILE)],
                                p1_out.at[pl.ds(row0, ROWS_PER_TILE)], st_s)
      b = pltpu.make_async_copy(dacc.at[pl.ds(row0, ROWS_PER_TILE)],
                                d1_out.at[pl.ds(row0, ROWS_PER_TILE)], st_m)
      a.start()
      b.start()
      a.wait()
      b.wait()

  return agg(x, src_idx, dst_idx, edge_weight)


BLK = 2000


def _tc_combine(x, p0, p1, d0, d1, W_self, W_neigh, bias):
  def body(x_ref, p0_ref, p1_ref, d0_ref, d1_ref, ws_ref, wn_ref, b_ref,
           o_ref):
    deg = d0_ref[...] + d1_ref[...]
    neigh = (p0_ref[...] + p1_ref[...]) / jnp.maximum(deg, 1.0)
    rst = (
        jnp.dot(x_ref[...], ws_ref[...], preferred_element_type=jnp.float32)
        + jnp.dot(neigh, wn_ref[...], preferred_element_type=jnp.float32)
        + b_ref[...]
    )
    o_ref[...] = jnp.where(rst >= 0, rst, 0.01 * rst)

  return pl.pallas_call(
      body,
      grid=(N_NODES // BLK,),
      in_specs=[
          pl.BlockSpec((BLK, D), lambda i: (i, 0)),
          pl.BlockSpec((BLK, D), lambda i: (i, 0)),
          pl.BlockSpec((BLK, D), lambda i: (i, 0)),
          pl.BlockSpec((BLK, 1), lambda i: (i, 0)),
          pl.BlockSpec((BLK, 1), lambda i: (i, 0)),
          pl.BlockSpec((D, D), lambda i: (0, 0)),
          pl.BlockSpec((D, D), lambda i: (0, 0)),
          pl.BlockSpec((1, D), lambda i: (0, 0)),
      ],
      out_specs=pl.BlockSpec((BLK, D), lambda i: (i, 0)),
      out_shape=jax.ShapeDtypeStruct((N_NODES, D), jnp.float32),
  )(x, p0, p1, d0, d1, W_self, W_neigh, bias)


def kernel(node_embeddings, edge_index, edge_weight, W_self, W_neigh, bias):
  flat = edge_index.astype(jnp.int32).reshape(2 * N_EDGES)
  w = edge_weight.astype(jnp.float32)

  p0, p1, d0, d1 = _sc_aggregate(node_embeddings, flat, flat, w)

  return _tc_combine(
      node_embeddings, p0, p1,
      d0.reshape(N_PAD, 1), d1.reshape(N_PAD, 1),
      W_self, W_neigh, bias.reshape(1, D),
  )
